# Initial kernel scaffold; baseline (speedup 1.0000x reference)
import jax
import jax.numpy as jnp
from jax.experimental import pallas as pl


def kernel(x, edge_index, W1l, W1r, b1, W2l, W2r, b2, Wih_f, Whh_f, bih_f, bhh_f, Wih_b, Whh_b, bih_b, bhh_b, Wa, ba, Wjk, bjk, Wl1, bl1, Wl2, bl2):
    raise NotImplementedError("write your pallas kernel here")


# speedup vs baseline: 2.7597x; 2.7597x over previous
import functools

import jax
import jax.numpy as jnp
from jax import lax
from jax.experimental import pallas as pl
from jax.experimental.pallas import tpu as pltpu
from jax.experimental.pallas import tpu_sc as plsc

_N = 50000
_E = 800000
_D = 100
_H = 64

_NC = 2
_NS = 16
_L = 16

_NHALF = _N // _NC
_NDUMMY = 8
_RPT = 1568
_ACC_ROWS = _RPT * _NS
_RPT_LAST = _NHALF - (_NS - 1) * _RPT
_K = 80
_EPT = _E // _NS
_NCHUNK = _EPT // _K

_BLK = 2000


def _local_idx(dstv, liv, base, dummy):
    for i in range(_K // _L):
        d = dstv[pl.ds(i * _L, _L)]
        inr = jnp.logical_and(d >= base, d < base + _NHALF)
        liv[pl.ds(i * _L, _L)] = jnp.where(inr, d - base, dummy)


def _out_copy(acc, out_hbm, sid, r0, o0):
    @pl.when(sid < _NS - 1)
    def _():
        pltpu.sync_copy(acc.at[pl.ds(r0, _RPT)], out_hbm.at[pl.ds(o0, _RPT)])

    @pl.when(sid == _NS - 1)
    def _():
        pltpu.sync_copy(acc.at[pl.ds(r0, _RPT_LAST)],
                        out_hbm.at[pl.ds(o0, _RPT_LAST)])


def _segsum_body(y_hbm, src_hbm, dst_hbm, z64_hbm, out_hbm,
                 acc, srcv, dstv, liv, rows, sem):
    cid = lax.axis_index("c")
    sid = lax.axis_index("s")
    base = pl.multiple_of(cid * _NHALF, 8)
    r0 = pl.multiple_of(sid * _RPT, 8)

    pltpu.sync_copy(z64_hbm, acc.at[pl.ds(r0, _RPT)])
    plsc.subcore_barrier()

    e0 = pl.multiple_of(sid * _EPT, 8)
    lane = lax.iota(jnp.int32, _L)
    dummy = _NHALF + (lane & (_NDUMMY - 1))

    @pl.loop(0, _NCHUNK)
    def _chunk(j):
        off = pl.multiple_of(e0 + j * _K, 8)
        pltpu.sync_copy(src_hbm.at[pl.ds(off, _K)], srcv)
        pltpu.sync_copy(dst_hbm.at[pl.ds(off, _K)], dstv)
        _local_idx(dstv, liv, base, dummy)
        pltpu.async_copy(y_hbm.at[srcv], rows, sem).wait()
        pltpu.sync_copy(rows, acc.at[liv], add=True)

    plsc.subcore_barrier()
    _out_copy(acc, out_hbm, sid, r0, pl.multiple_of(base + r0, 8))


def _deg_body(dst_hbm, z16_hbm, ones_hbm, deg_hbm,
              degacc, dstv, liv, ones):
    cid = lax.axis_index("c")
    sid = lax.axis_index("s")
    base = pl.multiple_of(cid * _NHALF, 8)
    r0 = pl.multiple_of(sid * _RPT, 8)

    pltpu.sync_copy(z16_hbm, degacc.at[pl.ds(r0, _RPT)])
    pltpu.sync_copy(ones_hbm, ones)
    plsc.subcore_barrier()

    e0 = pl.multiple_of(sid * _EPT, 8)
    lane = lax.iota(jnp.int32, _L)
    dummy = _NHALF + (lane & (_NDUMMY - 1))

    @pl.loop(0, _NCHUNK)
    def _chunk(j):
        off = pl.multiple_of(e0 + j * _K, 8)
        pltpu.sync_copy(dst_hbm.at[pl.ds(off, _K)], dstv)
        _local_idx(dstv, liv, base, dummy)
        pltpu.sync_copy(ones, degacc.at[liv], add=True)

    plsc.subcore_barrier()
    _out_copy(degacc, deg_hbm, sid, r0, pl.multiple_of(base + r0, 8))


def _sc_mesh():
    return plsc.VectorSubcoreMesh(core_axis_name="c", subcore_axis_name="s",
                                  num_cores=_NC, num_subcores=_NS)


def _make_segsum():
    scratch = [
        pltpu.VMEM_SHARED((_ACC_ROWS, _H), jnp.float32),
        pltpu.VMEM((_K,), jnp.int32),
        pltpu.VMEM((_K,), jnp.int32),
        pltpu.VMEM((_K,), jnp.int32),
        pltpu.VMEM((_K, _H), jnp.float32),
        pltpu.SemaphoreType.DMA,
    ]
    return pl.kernel(_segsum_body,
                     out_type=jax.ShapeDtypeStruct((_N, _H), jnp.float32),
                     mesh=_sc_mesh(), scratch_types=scratch,
                     compiler_params=pltpu.CompilerParams(
                         use_tc_tiling_on_sc=False))


def _make_deg():
    scratch = [
        pltpu.VMEM_SHARED((_ACC_ROWS, _L), jnp.float32),
        pltpu.VMEM((_K,), jnp.int32),
        pltpu.VMEM((_K,), jnp.int32),
        pltpu.VMEM((_K, _L), jnp.float32),
    ]
    return pl.kernel(_deg_body,
                     out_type=jax.ShapeDtypeStruct((_N, _L), jnp.float32),
                     mesh=_sc_mesh(), scratch_types=scratch,
                     compiler_params=pltpu.CompilerParams(
                         use_tc_tiling_on_sc=False))


def _pre_body(x_ref, wl_ref, wr_ref, b_ref, y_ref, r_ref):
    x = x_ref[...]
    y_ref[...] = jnp.dot(x, wl_ref[...], preferred_element_type=jnp.float32)
    r_ref[...] = jnp.dot(x, wr_ref[...],
                         preferred_element_type=jnp.float32) + b_ref[...]


def _mid_body(s_ref, deg_ref, r_ref, wl_ref, wr_ref, b_ref,
              x1_ref, y2_ref, r2_ref):
    dinv = 1.0 / jnp.maximum(deg_ref[:, 0:1], 1.0)
    x1 = jnp.maximum(s_ref[...] * dinv + r_ref[...], 0.0)
    x1_ref[...] = x1
    y2_ref[...] = jnp.dot(x1, wl_ref[...], preferred_element_type=jnp.float32)
    r2_ref[...] = jnp.dot(x1, wr_ref[...],
                          preferred_element_type=jnp.float32) + b_ref[...]


def _lstm_step(xt, h, c, wih_t, whh_t, bsum):
    g = jnp.dot(xt, wih_t, preferred_element_type=jnp.float32) + bsum
    if h is not None:
        g = g + jnp.dot(h, whh_t, preferred_element_type=jnp.float32)
    i = jax.nn.sigmoid(g[:, 0 * _H:1 * _H])
    f = jax.nn.sigmoid(g[:, 1 * _H:2 * _H])
    gg = jnp.tanh(g[:, 2 * _H:3 * _H])
    o = jax.nn.sigmoid(g[:, 3 * _H:4 * _H])
    c_new = i * gg if c is None else f * c + i * gg
    h_new = o * jnp.tanh(c_new)
    return h_new, c_new


def _post_body(s_ref, deg_ref, r_ref, x1_ref,
               wihf_ref, whhf_ref, bf_ref, wihb_ref, whhb_ref, bb_ref,
               wa_ref, ba_ref, wjk_ref, bjk_ref, wl1_ref, bl1_ref,
               wl2_ref, bl2_ref, out_ref):
    dinv = 1.0 / jnp.maximum(deg_ref[:, 0:1], 1.0)
    x2 = jnp.maximum(s_ref[...] * dinv + r_ref[...], 0.0)
    x1 = x1_ref[...]

    bf = bf_ref[...]
    bb = bb_ref[...]
    hf1, cf1 = _lstm_step(x1, None, None, wihf_ref[...], None, bf)
    hf2, _ = _lstm_step(x2, hf1, cf1, wihf_ref[...], whhf_ref[...], bf)
    hb1, cb1 = _lstm_step(x2, None, None, wihb_ref[...], None, bb)
    hb2, _ = _lstm_step(x1, hb1, cb1, wihb_ref[...], whhb_ref[...], bb)

    wa = wa_ref[...]
    ba = ba_ref[...]
    lout0 = jnp.concatenate([hf1, hb2], axis=1)
    lout1 = jnp.concatenate([hf2, hb1], axis=1)
    l0 = jnp.dot(lout0, wa, preferred_element_type=jnp.float32) + ba
    l1 = jnp.dot(lout1, wa, preferred_element_type=jnp.float32) + ba
    m = jnp.maximum(l0, l1)
    e0 = jnp.exp(l0 - m)
    e1 = jnp.exp(l1 - m)
    inv = 1.0 / (e0 + e1)
    xj = x1 * (e0 * inv) + x2 * (e1 * inv)
    xj = jnp.dot(xj, wjk_ref[...],
                 preferred_element_type=jnp.float32) + bjk_ref[...]
    h1 = jnp.maximum(
        jnp.dot(xj, wl1_ref[...],
                preferred_element_type=jnp.float32) + bl1_ref[...], 0.0)
    out_ref[...] = jnp.maximum(
        jnp.dot(h1, wl2_ref[...],
                preferred_element_type=jnp.float32) + bl2_ref[...], 0.0)


def _row_spec(cols):
    return pl.BlockSpec((_BLK, cols), lambda i: (i, 0))


def _full_spec(shape):
    return pl.BlockSpec(shape, lambda i: (0,) * len(shape))


def _tc_pre(x, wl, wr, b):
    grid = (_N // _BLK,)
    return pl.pallas_call(
        _pre_body,
        grid=grid,
        in_specs=[_row_spec(_D), _full_spec((_D, _H)), _full_spec((_D, _H)),
                  _full_spec((1, _H))],
        out_specs=[_row_spec(_H), _row_spec(_H)],
        out_shape=[jax.ShapeDtypeStruct((_N, _H), jnp.float32)] * 2,
    )(x, wl, wr, b)


def _tc_mid(s, deg, r, wl, wr, b):
    grid = (_N // _BLK,)
    return pl.pallas_call(
        _mid_body,
        grid=grid,
        in_specs=[_row_spec(_H), _row_spec(_L), _row_spec(_H),
                  _full_spec((_H, _H)), _full_spec((_H, _H)),
                  _full_spec((1, _H))],
        out_specs=[_row_spec(_H)] * 3,
        out_shape=[jax.ShapeDtypeStruct((_N, _H), jnp.float32)] * 3,
    )(s, deg, r, wl, wr, b)


def _tc_post(s, deg, r, x1, wihf, whhf, bf, wihb, whhb, bb,
             wa, ba, wjk, bjk, wl1, bl1, wl2, bl2):
    grid = (_N // _BLK,)
    return pl.pallas_call(
        _post_body,
        grid=grid,
        in_specs=[_row_spec(_H), _row_spec(_L), _row_spec(_H), _row_spec(_H),
                  _full_spec((_H, 4 * _H)), _full_spec((_H, 4 * _H)),
                  _full_spec((1, 4 * _H)),
                  _full_spec((_H, 4 * _H)), _full_spec((_H, 4 * _H)),
                  _full_spec((1, 4 * _H)),
                  _full_spec((2 * _H, 1)), _full_spec((1, 1)),
                  _full_spec((_H, _H)), _full_spec((1, _H)),
                  _full_spec((_H, _H)), _full_spec((1, _H)),
                  _full_spec((_H, 1)), _full_spec((1, 1))],
        out_specs=pl.BlockSpec((_BLK, 1), lambda i: (i, 0)),
        out_shape=jax.ShapeDtypeStruct((_N, 1), jnp.float32),
    )(s, deg, r, x1, wihf, whhf, bf, wihb, whhb, bb,
      wa, ba, wjk, bjk, wl1, bl1, wl2, bl2)


def kernel(x, edge_index, W1l, W1r, b1, W2l, W2r, b2,
           Wih_f, Whh_f, bih_f, bhh_f, Wih_b, Whh_b, bih_b, bhh_b,
           Wa, ba, Wjk, bjk, Wl1, bl1, Wl2, bl2):
    src = edge_index[0].astype(jnp.int32)
    dst = edge_index[1].astype(jnp.int32)
    z64 = jnp.zeros((_RPT, _H), jnp.float32)
    z16 = jnp.zeros((_RPT, _L), jnp.float32)
    ones = jnp.ones((_K, _L), jnp.float32)

    deg = _make_deg()(dst, z16, ones)
    y1, r1 = _tc_pre(x, W1l, W1r, b1.reshape(1, _H))
    s1 = _make_segsum()(y1, src, dst, z64)
    x1, y2, r2 = _tc_mid(s1, deg, r1, W2l, W2r, b2.reshape(1, _H))
    s2 = _make_segsum()(y2, src, dst, z64)
    out = _tc_post(
        s2, deg, r2, x1,
        Wih_f.T, Whh_f.T, (bih_f + bhh_f).reshape(1, 4 * _H),
        Wih_b.T, Whh_b.T, (bih_b + bhh_b).reshape(1, 4 * _H),
        Wa, ba.reshape(1, 1), Wjk, bjk.reshape(1, _H),
        Wl1, bl1.reshape(1, _H), Wl2, bl2.reshape(1, 1))
    return out
